# baseline (speedup 1.0000x reference)
import functools

import jax
import jax.numpy as jnp
from jax import lax
from jax.experimental import pallas as pl
from jax.experimental.pallas import tpu as pltpu
from jax.experimental.pallas import tpu_sc as plsc

B_, N_, T_, E_ = 4, 512, 1024, 32768
D_IN_, D_EDGE_IN_, D_NODE_, D_EDGE_ = 8, 8, 128, 16

NC, NS, L = 2, 16, 16
CELLS = B_ * N_ * N_
HALF = CELLS // NC
NPASS = 16
CHUNK = HALF // NPASS
RPT = CHUNK // NS
EPT = E_ // NS
NDUMP = 1024
VPT = EPT // L
ZR = 512
Z1 = 4096
CPT = HALF // NS


def _lane():
    return lax.iota(jnp.int32, L)


_sc_mesh = plsc.VectorSubcoreMesh(core_axis_name="c", subcore_axis_name="s")


@functools.partial(
    pl.kernel,
    mesh=_sc_mesh,
    compiler_params=pltpu.CompilerParams(needs_layout_passes=False,
                                         use_tc_tiling_on_sc=False),
    out_type=[jax.ShapeDtypeStruct((CELLS, 16), jnp.float32),
              jax.ShapeDtypeStruct((CELLS,), jnp.float32)],
    scratch_types=[
        pltpu.VMEM((EPT,), jnp.int32),
        pltpu.VMEM((EPT,), jnp.int32),
        pltpu.VMEM((EPT, 16), jnp.float32),
        pltpu.VMEM((T_,), jnp.int32),
        pltpu.VMEM((EPT,), jnp.int32),
        pltpu.VMEM((EPT,), jnp.float32),
        pltpu.VMEM((EPT // 128, 128), jnp.int32),
        pltpu.VMEM((ZR, 16), jnp.float32),
        pltpu.VMEM((Z1,), jnp.float32),
        pltpu.VMEM_SHARED((CHUNK + NDUMP, 16), jnp.float32),
        pltpu.VMEM_SHARED((HALF + NDUMP,), jnp.float32),
        pltpu.SemaphoreType.DMA,
    ],
)
def _sc_scatter(src_hbm, dst_hbm, rows_hbm, bid_hbm, acc_hbm, cnt_hbm,
                src_v, dst_v, row_v, bid_v, cell_v, ones_v, idx_v, zrow_v,
                z1d_v, a_sh, c_sh, dsem):
    c = lax.axis_index("c")
    s = lax.axis_index("s")
    lane = _lane()
    zero16f = jnp.zeros((L,), jnp.float32)

    e0 = s * EPT
    pltpu.sync_copy(src_hbm.at[pl.ds(e0, EPT)], src_v)
    pltpu.sync_copy(dst_hbm.at[pl.ds(e0, EPT)], dst_v)
    pltpu.sync_copy(rows_hbm.at[pl.ds(e0, EPT), :], row_v)
    pltpu.sync_copy(bid_hbm, bid_v)

    def zfill(j, _):
        zrow_v[j, :] = zero16f
        return 0

    lax.fori_loop(0, ZR, zfill, 0)

    def zfill1(j, _):
        plsc.store_scatter(z1d_v, [j * L + lane], zero16f)
        return 0

    lax.fori_loop(0, Z1 // L, zfill1, 0)

    def onesfill(j, _):
        plsc.store_scatter(ones_v, [j * L + lane], zero16f + 1.0)
        return 0

    lax.fori_loop(0, EPT // L, onesfill, 0)

    def obody(j, acc):
        v = plsc.load_gather(bid_v, [j * L + lane])
        a1, a2, a3 = acc
        return (a1 + (v < 1).astype(jnp.int32),
                a2 + (v < 2).astype(jnp.int32),
                a3 + (v < 3).astype(jnp.int32))

    z16 = jnp.zeros((L,), jnp.int32)
    a1, a2, a3 = lax.fori_loop(0, T_ // L, obody, (z16, z16, z16))
    off1 = jnp.sum(a1)
    off2 = jnp.sum(a2)
    off3 = jnp.sum(a3)

    def cbody(j, _):
        ii = j * L + lane
        sv = plsc.load_gather(src_v, [ii])
        dv = plsc.load_gather(dst_v, [ii])
        bs = ((sv >= off1).astype(jnp.int32) + (sv >= off2).astype(jnp.int32)
              + (sv >= off3).astype(jnp.int32))
        bd = ((dv >= off1).astype(jnp.int32) + (dv >= off2).astype(jnp.int32)
              + (dv >= off3).astype(jnp.int32))
        os_ = ((bs == 1).astype(jnp.int32) * off1 + (bs == 2).astype(jnp.int32) * off2
               + (bs == 3).astype(jnp.int32) * off3)
        od_ = ((bd == 1).astype(jnp.int32) * off1 + (bd == 2).astype(jnp.int32) * off2
               + (bd == 3).astype(jnp.int32) * off3)
        sl = sv - os_
        dl = dv - od_
        ok = (bs == bd) & (sl < N_) & (dl < N_)
        cell = jnp.where(ok, bs * (N_ * N_) + sl * N_ + dl, 1 << 30)
        plsc.store_scatter(cell_v, [ii], cell)
        return 0

    lax.fori_loop(0, VPT, cbody, 0)

    half0 = c * HALF

    def czero(z, _):
        pltpu.sync_copy(z1d_v, c_sh.at[pl.ds(s * CPT + z * Z1, Z1)])
        return 0

    lax.fori_loop(0, CPT // Z1, czero, 0)
    plsc.subcore_barrier()

    def crbody(r, _):
        def cibody(q, _):
            g = r * 128 + q * L
            cell = plsc.load_gather(cell_v, [g + lane])
            loc = cell - half0
            ok = (loc >= 0) & (loc < HALF)
            dump = HALF + ((g + lane) & (NDUMP - 1))
            idx = jnp.where(ok, loc, dump)
            plsc.store_scatter(idx_v, [lane * 0 + r, q * L + lane], idx)
            return 0

        lax.fori_loop(0, 128 // L, cibody, 0)
        pltpu.async_copy(ones_v.at[pl.ds(r * 128, 128)],
                         c_sh.at[idx_v.at[r]], dsem, add=True)
        return 0

    lax.fori_loop(0, EPT // 128, crbody, 0)

    def cdrain(r, _):
        pltpu.make_async_copy(ones_v.at[pl.ds(0, 128)],
                              c_sh.at[idx_v.at[0]], dsem).wait()
        return 0

    lax.fori_loop(0, EPT // 128, cdrain, 0)
    plsc.subcore_barrier()
    pltpu.sync_copy(c_sh.at[pl.ds(s * CPT, CPT)],
                    cnt_hbm.at[pl.ds(half0 + s * CPT, CPT)])

    def pbody(p, _):
        base = half0 + p * CHUNK

        def azero(z, _):
            pltpu.sync_copy(zrow_v, a_sh.at[pl.ds(s * RPT + z * ZR, ZR), :])
            return 0

        lax.fori_loop(0, RPT // ZR, azero, 0)
        plsc.subcore_barrier()

        def rbody(r, _):
            def ibody(q, _):
                g = r * 128 + q * L
                ii = g + lane
                cell = plsc.load_gather(cell_v, [ii])
                loc = cell - base
                ok = (loc >= 0) & (loc < CHUNK)
                dump = CHUNK + ((g + lane) & (NDUMP - 1))
                idx = jnp.where(ok, loc, dump)
                plsc.store_scatter(idx_v, [lane * 0 + r, q * L + lane], idx)
                return 0

            lax.fori_loop(0, 128 // L, ibody, 0)
            pltpu.async_copy(row_v.at[pl.ds(r * 128, 128), :],
                             a_sh.at[idx_v.at[r]], dsem, add=True)
            return 0

        lax.fori_loop(0, EPT // 128, rbody, 0)

        def rdrain(r, _):
            pltpu.make_async_copy(row_v.at[pl.ds(0, 128), :],
                                  a_sh.at[idx_v.at[0]], dsem).wait()
            return 0

        lax.fori_loop(0, EPT // 128, rdrain, 0)
        plsc.subcore_barrier()
        pltpu.sync_copy(a_sh.at[pl.ds(s * RPT, RPT), :],
                        acc_hbm.at[pl.ds(base + s * RPT, RPT), :])
        plsc.subcore_barrier()
        return 0

    lax.fori_loop(0, NPASS, pbody, 0)


CB = 16384


def _edge_body(a_ref, ww_ref, bw_ref, we_ref, be_ref, eh_ref):
    blk = a_ref[...]
    m_in = jnp.concatenate(
        [ww_ref[...], bw_ref[...][None, :], jnp.zeros((7, 16), jnp.float32)], axis=0)
    m = jnp.dot(m_in, we_ref[...], preferred_element_type=jnp.float32)
    z16 = jnp.zeros((16, 16), jnp.float32)
    mbig = jnp.concatenate(
        [jnp.concatenate([m if i == j else z16 for j in range(8)], axis=1)
         for i in range(8)], axis=0)
    ri = lax.broadcasted_iota(jnp.int32, (128, 128), 0)
    ci = lax.broadcasted_iota(jnp.int32, (128, 128), 1)
    sbig = ((ri % 16 == 8) & (ri // 16 == ci // 16)).astype(jnp.float32)
    be_t = jnp.concatenate([be_ref[...][None, :]] * 8, axis=1)
    prod = jnp.dot(blk, mbig, preferred_element_type=jnp.float32)
    cnt = jnp.dot(blk, sbig, preferred_element_type=jnp.float32)
    eh_ref[...] = (prod + be_t) * cnt


CB8 = 4096


def _edge_dense(acc2, W_weight, b_weight, W_edge, b_edge):
    grid = (CELLS // 8 // CB8,)
    return pl.pallas_call(
        _edge_body,
        grid=grid,
        in_specs=[
            pl.BlockSpec((CB8, 128), lambda i: (i, 0)),
            pl.BlockSpec((8, 16), lambda i: (0, 0)),
            pl.BlockSpec((16,), lambda i: (0,)),
            pl.BlockSpec((16, 16), lambda i: (0, 0)),
            pl.BlockSpec((16,), lambda i: (0,)),
        ],
        out_specs=pl.BlockSpec((CB8, 128), lambda i: (i, 0)),
        out_shape=jax.ShapeDtypeStruct((CELLS // 8, 128), jnp.float32),
    )(acc2, W_weight, b_weight, W_edge, b_edge)


def _node_body(x_ref, bid_ref, wb_ref, bb_ref, wn_ref, bn_ref, h_ref, m_ref):
    bid = bid_ref[...]
    offs = [0,
            jnp.sum((bid < 1).astype(jnp.int32)),
            jnp.sum((bid < 2).astype(jnp.int32)),
            jnp.sum((bid < 3).astype(jnp.int32)),
            T_]
    wc = jnp.dot(wb_ref[...], wn_ref[...], preferred_element_type=jnp.float32)
    bc = (jnp.dot(bb_ref[...][None, :], wn_ref[...],
                  preferred_element_type=jnp.float32) + bn_ref[...][None, :])
    for b in range(B_):
        cntb = offs[b + 1] - offs[b]
        xs = x_ref[pl.ds(offs[b], N_), pl.ds(0, D_IN_)]
        rows = lax.broadcasted_iota(jnp.int32, (N_, D_NODE_), 0)
        mk = (rows < cntb).astype(jnp.float32)
        h_ref[pl.ds(b * N_, N_), pl.ds(0, D_NODE_)] = (
            jnp.dot(xs, wc, preferred_element_type=jnp.float32) + bc) * mk
        col = lax.broadcasted_iota(jnp.int32, (1, N_), 1)
        m_ref[pl.ds(b, 1), pl.ds(0, N_)] = (col < cntb).astype(jnp.int32)


def _node_dense(x_pad, batch_ids, W_bias, b_bias, W_node, b_node):
    return pl.pallas_call(
        _node_body,
        out_shape=[
            jax.ShapeDtypeStruct((B_ * N_, D_NODE_), jnp.float32),
            jax.ShapeDtypeStruct((B_, N_), jnp.int32),
        ],
    )(x_pad, batch_ids, W_bias, b_bias, W_node, b_node)


def kernel(x, edge_index, edge_attr, batch_ids, W_bias, b_bias, W_node, b_node,
           W_weight, b_weight, W_edge, b_edge):
    src = edge_index[0]
    dst = edge_index[1]
    rows16 = jnp.concatenate(
        [edge_attr,
         jnp.ones((E_, 1), jnp.float32),
         jnp.zeros((E_, 7), jnp.float32)], axis=1)
    acc, adj_flat = _sc_scatter(src, dst, rows16, batch_ids)
    eh_flat = _edge_dense(acc.reshape(CELLS // 8, 128),
                          W_weight, b_weight, W_edge, b_edge)
    x_pad = jnp.pad(x, ((0, N_), (0, 0)))
    h_flat, mask_i = _node_dense(x_pad, batch_ids, W_bias, b_bias, W_node, b_node)
    h = h_flat.reshape(B_, N_, D_NODE_)
    adj = adj_flat.reshape(B_, N_, N_)
    eh = eh_flat.reshape(B_, N_, N_, D_EDGE_)
    node_mask = mask_i.astype(jnp.bool_)
    return h, adj, eh, node_mask


# speedup vs baseline: 1.0279x; 1.0279x over previous
import functools

import jax
import jax.numpy as jnp
from jax import lax
from jax.experimental import pallas as pl
from jax.experimental.pallas import tpu as pltpu
from jax.experimental.pallas import tpu_sc as plsc

B_, N_, T_, E_ = 4, 512, 1024, 32768
D_IN_, D_EDGE_IN_, D_NODE_, D_EDGE_ = 8, 8, 128, 16

NC, NS, L = 2, 16, 16
CELLS = B_ * N_ * N_
HALF = CELLS // NC
NPASS = 32
CHUNK = HALF // NPASS
RPT = CHUNK // NS
EPT = E_ // NS
NDUMP = 1024
VPT = EPT // L
ZR = 512
Z1 = 4096
CPT = HALF // NS


def _lane():
    return lax.iota(jnp.int32, L)


_sc_mesh = plsc.VectorSubcoreMesh(core_axis_name="c", subcore_axis_name="s")


@functools.partial(
    pl.kernel,
    mesh=_sc_mesh,
    compiler_params=pltpu.CompilerParams(needs_layout_passes=False,
                                         use_tc_tiling_on_sc=False),
    out_type=[jax.ShapeDtypeStruct((CELLS, 16), jnp.float32),
              jax.ShapeDtypeStruct((CELLS,), jnp.float32)],
    scratch_types=[
        pltpu.VMEM((EPT,), jnp.int32),
        pltpu.VMEM((EPT,), jnp.int32),
        pltpu.VMEM((EPT, 16), jnp.float32),
        pltpu.VMEM((T_,), jnp.int32),
        pltpu.VMEM((EPT,), jnp.int32),
        pltpu.VMEM((EPT,), jnp.float32),
        pltpu.VMEM((EPT // 128, 128), jnp.int32),
        pltpu.VMEM((ZR, 16), jnp.float32),
        pltpu.VMEM((Z1,), jnp.float32),
        pltpu.VMEM_SHARED((2 * (CHUNK + NDUMP), 16), jnp.float32),
        pltpu.VMEM_SHARED((HALF + NDUMP,), jnp.float32),
        pltpu.SemaphoreType.DMA,
        pltpu.SemaphoreType.DMA,
        pltpu.SemaphoreType.DMA,
    ],
)
def _sc_scatter(src_hbm, dst_hbm, rows_hbm, bid_hbm, acc_hbm, cnt_hbm,
                src_v, dst_v, row_v, bid_v, cell_v, ones_v, idx_v, zrow_v,
                z1d_v, a_sh, c_sh, dsem, zsem, osem):
    c = lax.axis_index("c")
    s = lax.axis_index("s")
    lane = _lane()
    zero16f = jnp.zeros((L,), jnp.float32)

    e0 = s * EPT
    pltpu.sync_copy(src_hbm.at[pl.ds(e0, EPT)], src_v)
    pltpu.sync_copy(dst_hbm.at[pl.ds(e0, EPT)], dst_v)
    pltpu.sync_copy(rows_hbm.at[pl.ds(e0, EPT), :], row_v)
    pltpu.sync_copy(bid_hbm, bid_v)

    def zfill(j, _):
        zrow_v[j, :] = zero16f
        return 0

    lax.fori_loop(0, ZR, zfill, 0)

    def zfill1(j, _):
        plsc.store_scatter(z1d_v, [j * L + lane], zero16f)
        return 0

    lax.fori_loop(0, Z1 // L, zfill1, 0)

    def onesfill(j, _):
        plsc.store_scatter(ones_v, [j * L + lane], zero16f + 1.0)
        return 0

    lax.fori_loop(0, EPT // L, onesfill, 0)

    def obody(j, acc):
        v = plsc.load_gather(bid_v, [j * L + lane])
        a1, a2, a3 = acc
        return (a1 + (v < 1).astype(jnp.int32),
                a2 + (v < 2).astype(jnp.int32),
                a3 + (v < 3).astype(jnp.int32))

    z16 = jnp.zeros((L,), jnp.int32)
    a1, a2, a3 = lax.fori_loop(0, T_ // L, obody, (z16, z16, z16))
    off1 = jnp.sum(a1)
    off2 = jnp.sum(a2)
    off3 = jnp.sum(a3)

    def cbody(j, _):
        ii = j * L + lane
        sv = plsc.load_gather(src_v, [ii])
        dv = plsc.load_gather(dst_v, [ii])
        bs = ((sv >= off1).astype(jnp.int32) + (sv >= off2).astype(jnp.int32)
              + (sv >= off3).astype(jnp.int32))
        bd = ((dv >= off1).astype(jnp.int32) + (dv >= off2).astype(jnp.int32)
              + (dv >= off3).astype(jnp.int32))
        os_ = ((bs == 1).astype(jnp.int32) * off1 + (bs == 2).astype(jnp.int32) * off2
               + (bs == 3).astype(jnp.int32) * off3)
        od_ = ((bd == 1).astype(jnp.int32) * off1 + (bd == 2).astype(jnp.int32) * off2
               + (bd == 3).astype(jnp.int32) * off3)
        sl = sv - os_
        dl = dv - od_
        ok = (bs == bd) & (sl < N_) & (dl < N_)
        cell = jnp.where(ok, bs * (N_ * N_) + sl * N_ + dl, 1 << 30)
        plsc.store_scatter(cell_v, [ii], cell)
        return 0

    lax.fori_loop(0, VPT, cbody, 0)

    half0 = c * HALF

    def czero(z, _):
        pltpu.sync_copy(z1d_v, c_sh.at[pl.ds(s * CPT + z * Z1, Z1)])
        return 0

    lax.fori_loop(0, CPT // Z1, czero, 0)
    plsc.subcore_barrier()

    def crbody(r, _):
        def cibody(q, _):
            g = r * 128 + q * L
            cell = plsc.load_gather(cell_v, [g + lane])
            loc = cell - half0
            ok = (loc >= 0) & (loc < HALF)
            dump = HALF + ((g + lane) & (NDUMP - 1))
            idx = jnp.where(ok, loc, dump)
            plsc.store_scatter(idx_v, [lane * 0 + r, q * L + lane], idx)
            return 0

        lax.fori_loop(0, 128 // L, cibody, 0)
        pltpu.async_copy(ones_v.at[pl.ds(r * 128, 128)],
                         c_sh.at[idx_v.at[r]], dsem, add=True)
        return 0

    lax.fori_loop(0, EPT // 128, crbody, 0)

    def cdrain(r, _):
        pltpu.make_async_copy(ones_v.at[pl.ds(0, 128)],
                              c_sh.at[idx_v.at[0]], dsem).wait()
        return 0

    lax.fori_loop(0, EPT // 128, cdrain, 0)
    plsc.subcore_barrier()
    pltpu.sync_copy(c_sh.at[pl.ds(s * CPT, CPT)],
                    cnt_hbm.at[pl.ds(half0 + s * CPT, CPT)])

    def pbody(p, _):
        base = half0 + p * CHUNK
        boff = lax.rem(p, 2) * (CHUNK + NDUMP)

        @pl.when(p >= 2)
        def _():
            pltpu.make_async_copy(
                a_sh.at[pl.ds(boff + s * RPT, RPT), :],
                acc_hbm.at[pl.ds(base - 2 * CHUNK + s * RPT, RPT), :],
                osem).wait()

        pltpu.async_copy(zrow_v, a_sh.at[pl.ds(boff + s * RPT, ZR), :], zsem)
        pltpu.async_copy(zrow_v, a_sh.at[pl.ds(boff + s * RPT + ZR, ZR), :],
                         zsem)

        def rbody(r, _):
            def ibody(q, _):
                g = r * 128 + q * L
                ii = g + lane
                cell = plsc.load_gather(cell_v, [ii])
                loc = cell - base
                ok = (loc >= 0) & (loc < CHUNK)
                dump = CHUNK + ((g + lane) & (NDUMP - 1))
                idx = boff + jnp.where(ok, loc, dump)
                plsc.store_scatter(idx_v, [lane * 0 + r, q * L + lane], idx)
                return 0

            lax.fori_loop(0, 128 // L, ibody, 0)
            return 0

        lax.fori_loop(0, EPT // 128, rbody, 0)
        pltpu.make_async_copy(zrow_v, a_sh.at[pl.ds(boff + s * RPT, ZR), :],
                              zsem).wait()
        pltpu.make_async_copy(zrow_v, a_sh.at[pl.ds(boff + s * RPT, ZR), :],
                              zsem).wait()
        plsc.subcore_barrier()

        def rfire(r, _):
            pltpu.async_copy(row_v.at[pl.ds(r * 128, 128), :],
                             a_sh.at[idx_v.at[r]], dsem, add=True)
            return 0

        lax.fori_loop(0, EPT // 128, rfire, 0)

        def rdrain(r, _):
            pltpu.make_async_copy(row_v.at[pl.ds(0, 128), :],
                                  a_sh.at[idx_v.at[0]], dsem).wait()
            return 0

        lax.fori_loop(0, EPT // 128, rdrain, 0)
        plsc.subcore_barrier()
        pltpu.async_copy(a_sh.at[pl.ds(boff + s * RPT, RPT), :],
                         acc_hbm.at[pl.ds(base + s * RPT, RPT), :], osem)
        return 0

    lax.fori_loop(0, NPASS, pbody, 0)

    def odrain(r, _):
        pltpu.make_async_copy(a_sh.at[pl.ds(s * RPT, RPT), :],
                              acc_hbm.at[pl.ds(s * RPT, RPT), :], osem).wait()
        return 0

    lax.fori_loop(0, 2, odrain, 0)


CB = 16384


def _edge_body(a_ref, ww_ref, bw_ref, we_ref, be_ref, eh_ref):
    blk = a_ref[...]
    m_in = jnp.concatenate(
        [ww_ref[...], bw_ref[...][None, :], jnp.zeros((7, 16), jnp.float32)], axis=0)
    m = jnp.dot(m_in, we_ref[...], preferred_element_type=jnp.float32)
    z16 = jnp.zeros((16, 16), jnp.float32)
    mbig = jnp.concatenate(
        [jnp.concatenate([m if i == j else z16 for j in range(8)], axis=1)
         for i in range(8)], axis=0)
    ri = lax.broadcasted_iota(jnp.int32, (128, 128), 0)
    ci = lax.broadcasted_iota(jnp.int32, (128, 128), 1)
    sbig = ((ri % 16 == 8) & (ri // 16 == ci // 16)).astype(jnp.float32)
    be_t = jnp.concatenate([be_ref[...][None, :]] * 8, axis=1)
    prod = jnp.dot(blk, mbig, preferred_element_type=jnp.float32)
    cnt = jnp.dot(blk, sbig, preferred_element_type=jnp.float32)
    eh_ref[...] = (prod + be_t) * cnt


CB8 = 4096


def _edge_dense(acc2, W_weight, b_weight, W_edge, b_edge):
    grid = (CELLS // 8 // CB8,)
    return pl.pallas_call(
        _edge_body,
        grid=grid,
        in_specs=[
            pl.BlockSpec((CB8, 128), lambda i: (i, 0)),
            pl.BlockSpec((8, 16), lambda i: (0, 0)),
            pl.BlockSpec((16,), lambda i: (0,)),
            pl.BlockSpec((16, 16), lambda i: (0, 0)),
            pl.BlockSpec((16,), lambda i: (0,)),
        ],
        out_specs=pl.BlockSpec((CB8, 128), lambda i: (i, 0)),
        out_shape=jax.ShapeDtypeStruct((CELLS // 8, 128), jnp.float32),
    )(acc2, W_weight, b_weight, W_edge, b_edge)


def _node_body(x_ref, bid_ref, wb_ref, bb_ref, wn_ref, bn_ref, h_ref, m_ref):
    bid = bid_ref[...]
    offs = [0,
            jnp.sum((bid < 1).astype(jnp.int32)),
            jnp.sum((bid < 2).astype(jnp.int32)),
            jnp.sum((bid < 3).astype(jnp.int32)),
            T_]
    wc = jnp.dot(wb_ref[...], wn_ref[...], preferred_element_type=jnp.float32)
    bc = (jnp.dot(bb_ref[...][None, :], wn_ref[...],
                  preferred_element_type=jnp.float32) + bn_ref[...][None, :])
    for b in range(B_):
        cntb = offs[b + 1] - offs[b]
        xs = x_ref[pl.ds(offs[b], N_), pl.ds(0, D_IN_)]
        rows = lax.broadcasted_iota(jnp.int32, (N_, D_NODE_), 0)
        mk = (rows < cntb).astype(jnp.float32)
        h_ref[pl.ds(b * N_, N_), pl.ds(0, D_NODE_)] = (
            jnp.dot(xs, wc, preferred_element_type=jnp.float32) + bc) * mk
        col = lax.broadcasted_iota(jnp.int32, (1, N_), 1)
        m_ref[pl.ds(b, 1), pl.ds(0, N_)] = (col < cntb).astype(jnp.int32)


def _node_dense(x_pad, batch_ids, W_bias, b_bias, W_node, b_node):
    return pl.pallas_call(
        _node_body,
        out_shape=[
            jax.ShapeDtypeStruct((B_ * N_, D_NODE_), jnp.float32),
            jax.ShapeDtypeStruct((B_, N_), jnp.int32),
        ],
    )(x_pad, batch_ids, W_bias, b_bias, W_node, b_node)


def kernel(x, edge_index, edge_attr, batch_ids, W_bias, b_bias, W_node, b_node,
           W_weight, b_weight, W_edge, b_edge):
    src = edge_index[0]
    dst = edge_index[1]
    rows16 = jnp.concatenate(
        [edge_attr,
         jnp.ones((E_, 1), jnp.float32),
         jnp.zeros((E_, 7), jnp.float32)], axis=1)
    acc, adj_flat = _sc_scatter(src, dst, rows16, batch_ids)
    eh_flat = _edge_dense(acc.reshape(CELLS // 8, 128),
                          W_weight, b_weight, W_edge, b_edge)
    x_pad = jnp.pad(x, ((0, N_), (0, 0)))
    h_flat, mask_i = _node_dense(x_pad, batch_ids, W_bias, b_bias, W_node, b_node)
    h = h_flat.reshape(B_, N_, D_NODE_)
    adj = adj_flat.reshape(B_, N_, N_)
    eh = eh_flat.reshape(B_, N_, N_, D_EDGE_)
    node_mask = mask_i.astype(jnp.bool_)
    return h, adj, eh, node_mask


# speedup vs baseline: 1.0485x; 1.0200x over previous
import functools

import jax
import jax.numpy as jnp
from jax import lax
from jax.experimental import pallas as pl
from jax.experimental.pallas import tpu as pltpu
from jax.experimental.pallas import tpu_sc as plsc

B_, N_, T_, E_ = 4, 512, 1024, 32768
D_IN_, D_EDGE_IN_, D_NODE_, D_EDGE_ = 8, 8, 128, 16

NC, NS, L = 2, 16, 16
CELLS = B_ * N_ * N_
HALF = CELLS // NC
NPASS = 32
CHUNK = HALF // NPASS
RPT = CHUNK // NS
EPT = E_ // NS
NDUMP = 1024
VPT = EPT // L
ZR = 512
Z1 = 4096
CPT = HALF // NS


def _lane():
    return lax.iota(jnp.int32, L)


_sc_mesh = plsc.VectorSubcoreMesh(core_axis_name="c", subcore_axis_name="s")


@functools.partial(
    pl.kernel,
    mesh=_sc_mesh,
    compiler_params=pltpu.CompilerParams(needs_layout_passes=False,
                                         use_tc_tiling_on_sc=False),
    out_type=[jax.ShapeDtypeStruct((CELLS, 16), jnp.float32),
              jax.ShapeDtypeStruct((CELLS,), jnp.float32)],
    scratch_types=[
        pltpu.VMEM((EPT,), jnp.int32),
        pltpu.VMEM((EPT,), jnp.int32),
        pltpu.VMEM((EPT, 16), jnp.float32),
        pltpu.VMEM((T_,), jnp.int32),
        pltpu.VMEM((EPT,), jnp.int32),
        pltpu.VMEM((EPT,), jnp.float32),
        pltpu.VMEM((EPT // 128, 128), jnp.int32),
        pltpu.VMEM((ZR, 16), jnp.float32),
        pltpu.VMEM((Z1,), jnp.float32),
        pltpu.VMEM_SHARED((2 * (CHUNK + NDUMP), 16), jnp.float32),
        pltpu.VMEM_SHARED((HALF + NDUMP,), jnp.float32),
        pltpu.SemaphoreType.DMA,
        pltpu.SemaphoreType.DMA,
        pltpu.SemaphoreType.DMA,
        pltpu.SemaphoreType.DMA,
    ],
)
def _sc_scatter(src_hbm, dst_hbm, rows_hbm, bid_hbm, acc_hbm, cnt_hbm,
                src_v, dst_v, row_v, bid_v, cell_v, ones_v, idx_v, zrow_v,
                z1d_v, a_sh, c_sh, dsem, zsem, osem, csem):
    c = lax.axis_index("c")
    s = lax.axis_index("s")
    lane = _lane()
    zero16f = jnp.zeros((L,), jnp.float32)

    e0 = s * EPT
    pltpu.async_copy(src_hbm.at[pl.ds(e0, EPT)], src_v, osem)
    pltpu.async_copy(dst_hbm.at[pl.ds(e0, EPT)], dst_v, osem)
    pltpu.async_copy(rows_hbm.at[pl.ds(e0, EPT), :], row_v, osem)
    pltpu.async_copy(bid_hbm, bid_v, osem)

    def zfill(j, _):
        zrow_v[j, :] = zero16f
        return 0

    lax.fori_loop(0, ZR, zfill, 0)

    def zfill1(j, _):
        plsc.store_scatter(z1d_v, [j * L + lane], zero16f)
        return 0

    lax.fori_loop(0, Z1 // L, zfill1, 0)

    def onesfill(j, _):
        plsc.store_scatter(ones_v, [j * L + lane], zero16f + 1.0)
        return 0

    lax.fori_loop(0, EPT // L, onesfill, 0)

    def czero(z, _):
        pltpu.async_copy(z1d_v, c_sh.at[pl.ds(s * CPT + z * Z1, Z1)], zsem)
        return 0

    lax.fori_loop(0, CPT // Z1, czero, 0)

    pltpu.make_async_copy(src_hbm.at[pl.ds(e0, EPT)], src_v, osem).wait()
    pltpu.make_async_copy(dst_hbm.at[pl.ds(e0, EPT)], dst_v, osem).wait()
    pltpu.make_async_copy(rows_hbm.at[pl.ds(e0, EPT), :], row_v, osem).wait()
    pltpu.make_async_copy(bid_hbm, bid_v, osem).wait()

    def obody(j, acc):
        v = plsc.load_gather(bid_v, [j * L + lane])
        a1, a2, a3 = acc
        return (a1 + (v < 1).astype(jnp.int32),
                a2 + (v < 2).astype(jnp.int32),
                a3 + (v < 3).astype(jnp.int32))

    z16 = jnp.zeros((L,), jnp.int32)
    a1, a2, a3 = lax.fori_loop(0, T_ // L, obody, (z16, z16, z16))
    off1 = jnp.sum(a1)
    off2 = jnp.sum(a2)
    off3 = jnp.sum(a3)

    def cbody(j, _):
        ii = j * L + lane
        sv = plsc.load_gather(src_v, [ii])
        dv = plsc.load_gather(dst_v, [ii])
        bs = ((sv >= off1).astype(jnp.int32) + (sv >= off2).astype(jnp.int32)
              + (sv >= off3).astype(jnp.int32))
        bd = ((dv >= off1).astype(jnp.int32) + (dv >= off2).astype(jnp.int32)
              + (dv >= off3).astype(jnp.int32))
        os_ = ((bs == 1).astype(jnp.int32) * off1 + (bs == 2).astype(jnp.int32) * off2
               + (bs == 3).astype(jnp.int32) * off3)
        od_ = ((bd == 1).astype(jnp.int32) * off1 + (bd == 2).astype(jnp.int32) * off2
               + (bd == 3).astype(jnp.int32) * off3)
        sl = sv - os_
        dl = dv - od_
        ok = (bs == bd) & (sl < N_) & (dl < N_)
        cell = jnp.where(ok, bs * (N_ * N_) + sl * N_ + dl, 1 << 30)
        plsc.store_scatter(cell_v, [ii], cell)
        return 0

    lax.fori_loop(0, VPT, cbody, 0)

    half0 = c * HALF

    def czdrain(z, _):
        pltpu.make_async_copy(z1d_v, c_sh.at[pl.ds(s * CPT, Z1)], zsem).wait()
        return 0

    lax.fori_loop(0, CPT // Z1, czdrain, 0)
    plsc.subcore_barrier()

    def crbody(r, _):
        def cibody(q, _):
            g = r * 128 + q * L
            cell = plsc.load_gather(cell_v, [g + lane])
            loc = cell - half0
            ok = (loc >= 0) & (loc < HALF)
            dump = HALF + ((g + lane) & (NDUMP - 1))
            idx = jnp.where(ok, loc, dump)
            plsc.store_scatter(idx_v, [lane * 0 + r, q * L + lane], idx)
            return 0

        lax.fori_loop(0, 128 // L, cibody, 0)
        pltpu.async_copy(ones_v.at[pl.ds(r * 128, 128)],
                         c_sh.at[idx_v.at[r]], dsem, add=True)
        return 0

    lax.fori_loop(0, EPT // 128, crbody, 0)

    def cdrain(r, _):
        pltpu.make_async_copy(ones_v.at[pl.ds(0, 128)],
                              c_sh.at[idx_v.at[0]], dsem).wait()
        return 0

    lax.fori_loop(0, EPT // 128, cdrain, 0)
    plsc.subcore_barrier()
    pltpu.async_copy(c_sh.at[pl.ds(s * CPT, CPT)],
                     cnt_hbm.at[pl.ds(half0 + s * CPT, CPT)], csem)

    def pbody(p, _):
        base = half0 + p * CHUNK
        boff = lax.rem(p, 2) * (CHUNK + NDUMP)

        @pl.when(p >= 2)
        def _():
            pltpu.make_async_copy(
                a_sh.at[pl.ds(boff + s * RPT, RPT), :],
                acc_hbm.at[pl.ds(base - 2 * CHUNK + s * RPT, RPT), :],
                osem).wait()

        pltpu.async_copy(zrow_v, a_sh.at[pl.ds(boff + s * RPT, ZR), :], zsem)
        pltpu.async_copy(zrow_v, a_sh.at[pl.ds(boff + s * RPT + ZR, ZR), :],
                         zsem)

        def rbody(r, _):
            def ibody(q, _):
                g = r * 128 + q * L
                ii = g + lane
                cell = plsc.load_gather(cell_v, [ii])
                loc = cell - base
                ok = (loc >= 0) & (loc < CHUNK)
                dump = CHUNK + ((g + lane) & (NDUMP - 1))
                idx = boff + jnp.where(ok, loc, dump)
                plsc.store_scatter(idx_v, [lane * 0 + r, q * L + lane], idx)
                return 0

            lax.fori_loop(0, 128 // L, ibody, 0)
            return 0

        lax.fori_loop(0, EPT // 128, rbody, 0)
        pltpu.make_async_copy(zrow_v, a_sh.at[pl.ds(boff + s * RPT, ZR), :],
                              zsem).wait()
        pltpu.make_async_copy(zrow_v, a_sh.at[pl.ds(boff + s * RPT, ZR), :],
                              zsem).wait()
        plsc.subcore_barrier()

        def rfire(r, _):
            pltpu.async_copy(row_v.at[pl.ds(r * 128, 128), :],
                             a_sh.at[idx_v.at[r]], dsem, add=True)
            return 0

        lax.fori_loop(0, EPT // 128, rfire, 0)

        def rdrain(r, _):
            pltpu.make_async_copy(row_v.at[pl.ds(0, 128), :],
                                  a_sh.at[idx_v.at[0]], dsem).wait()
            return 0

        lax.fori_loop(0, EPT // 128, rdrain, 0)
        plsc.subcore_barrier()
        pltpu.async_copy(a_sh.at[pl.ds(boff + s * RPT, RPT), :],
                         acc_hbm.at[pl.ds(base + s * RPT, RPT), :], osem)
        return 0

    lax.fori_loop(0, NPASS, pbody, 0)

    def odrain(r, _):
        pltpu.make_async_copy(a_sh.at[pl.ds(s * RPT, RPT), :],
                              acc_hbm.at[pl.ds(s * RPT, RPT), :], osem).wait()
        return 0

    lax.fori_loop(0, 2, odrain, 0)
    pltpu.make_async_copy(c_sh.at[pl.ds(s * CPT, CPT)],
                          cnt_hbm.at[pl.ds(half0 + s * CPT, CPT)], csem).wait()


CB = 16384


def _edge_body(a_ref, ww_ref, bw_ref, we_ref, be_ref, eh_ref):
    blk = a_ref[...]
    m_in = jnp.concatenate(
        [ww_ref[...], bw_ref[...][None, :], jnp.zeros((7, 16), jnp.float32)], axis=0)
    m = jnp.dot(m_in, we_ref[...], preferred_element_type=jnp.float32)
    z16 = jnp.zeros((16, 16), jnp.float32)
    mbig = jnp.concatenate(
        [jnp.concatenate([m if i == j else z16 for j in range(8)], axis=1)
         for i in range(8)], axis=0)
    ri = lax.broadcasted_iota(jnp.int32, (128, 128), 0)
    ci = lax.broadcasted_iota(jnp.int32, (128, 128), 1)
    sbig = ((ri % 16 == 8) & (ri // 16 == ci // 16)).astype(jnp.float32)
    be_t = jnp.concatenate([be_ref[...][None, :]] * 8, axis=1)
    prod = jnp.dot(blk, mbig, preferred_element_type=jnp.float32)
    cnt = jnp.dot(blk, sbig, preferred_element_type=jnp.float32)
    eh_ref[...] = (prod + be_t) * cnt


CB8 = 4096


def _edge_dense(acc2, W_weight, b_weight, W_edge, b_edge):
    grid = (CELLS // 8 // CB8,)
    return pl.pallas_call(
        _edge_body,
        grid=grid,
        in_specs=[
            pl.BlockSpec((CB8, 128), lambda i: (i, 0)),
            pl.BlockSpec((8, 16), lambda i: (0, 0)),
            pl.BlockSpec((16,), lambda i: (0,)),
            pl.BlockSpec((16, 16), lambda i: (0, 0)),
            pl.BlockSpec((16,), lambda i: (0,)),
        ],
        out_specs=pl.BlockSpec((CB8, 128), lambda i: (i, 0)),
        out_shape=jax.ShapeDtypeStruct((CELLS // 8, 128), jnp.float32),
    )(acc2, W_weight, b_weight, W_edge, b_edge)


def _node_body(x_ref, bid_ref, wb_ref, bb_ref, wn_ref, bn_ref, h_ref, m_ref):
    bid = bid_ref[...]
    offs = [0,
            jnp.sum((bid < 1).astype(jnp.int32)),
            jnp.sum((bid < 2).astype(jnp.int32)),
            jnp.sum((bid < 3).astype(jnp.int32)),
            T_]
    wc = jnp.dot(wb_ref[...], wn_ref[...], preferred_element_type=jnp.float32)
    bc = (jnp.dot(bb_ref[...][None, :], wn_ref[...],
                  preferred_element_type=jnp.float32) + bn_ref[...][None, :])
    for b in range(B_):
        cntb = offs[b + 1] - offs[b]
        xs = x_ref[pl.ds(offs[b], N_), pl.ds(0, D_IN_)]
        rows = lax.broadcasted_iota(jnp.int32, (N_, D_NODE_), 0)
        mk = (rows < cntb).astype(jnp.float32)
        h_ref[pl.ds(b * N_, N_), pl.ds(0, D_NODE_)] = (
            jnp.dot(xs, wc, preferred_element_type=jnp.float32) + bc) * mk
        col = lax.broadcasted_iota(jnp.int32, (1, N_), 1)
        m_ref[pl.ds(b, 1), pl.ds(0, N_)] = (col < cntb).astype(jnp.int32)


def _node_dense(x_pad, batch_ids, W_bias, b_bias, W_node, b_node):
    return pl.pallas_call(
        _node_body,
        out_shape=[
            jax.ShapeDtypeStruct((B_ * N_, D_NODE_), jnp.float32),
            jax.ShapeDtypeStruct((B_, N_), jnp.int32),
        ],
    )(x_pad, batch_ids, W_bias, b_bias, W_node, b_node)


def kernel(x, edge_index, edge_attr, batch_ids, W_bias, b_bias, W_node, b_node,
           W_weight, b_weight, W_edge, b_edge):
    src = edge_index[0]
    dst = edge_index[1]
    rows16 = jnp.concatenate(
        [edge_attr,
         jnp.ones((E_, 1), jnp.float32),
         jnp.zeros((E_, 7), jnp.float32)], axis=1)
    acc, adj_flat = _sc_scatter(src, dst, rows16, batch_ids)
    eh_flat = _edge_dense(acc.reshape(CELLS // 8, 128),
                          W_weight, b_weight, W_edge, b_edge)
    x_pad = jnp.pad(x, ((0, N_), (0, 0)))
    h_flat, mask_i = _node_dense(x_pad, batch_ids, W_bias, b_bias, W_node, b_node)
    h = h_flat.reshape(B_, N_, D_NODE_)
    adj = adj_flat.reshape(B_, N_, N_)
    eh = eh_flat.reshape(B_, N_, N_, D_EDGE_)
    node_mask = mask_i.astype(jnp.bool_)
    return h, adj, eh, node_mask
